# baseline (speedup 1.0000x reference)
import jax
import jax.numpy as jnp
from jax.experimental import pallas as pl
from jax.experimental.pallas import tpu as pltpu

_BM = 2048
_BN = 512


def _mm_body(lhs_ref, rhs_ref, out_ref):
    out_ref[...] = jnp.dot(
        lhs_ref[...], rhs_ref[...], preferred_element_type=jnp.float32
    )


def kernel(lhs, rhs):
    M, K = lhs.shape
    _, N = rhs.shape
    lhs_bf = lhs.astype(jnp.bfloat16)
    rhs_bf = rhs.astype(jnp.bfloat16)

    grid = (M // _BM, N // _BN)
    cost = pl.CostEstimate(
        flops=2 * M * N * K,
        transcendentals=0,
        bytes_accessed=(M * K + K * N) * 2 + M * N * 4,
    )
    return pl.pallas_call(
        _mm_body,
        out_shape=jax.ShapeDtypeStruct((M, N), jnp.float32),
        grid=grid,
        in_specs=[
            pl.BlockSpec((_BM, K), lambda i, j: (i, 0)),
            pl.BlockSpec((K, _BN), lambda i, j: (0, j)),
        ],
        out_specs=pl.BlockSpec((_BM, _BN), lambda i, j: (i, j)),
        compiler_params=pltpu.CompilerParams(
            dimension_semantics=("parallel", "parallel"),
            vmem_limit_bytes=60 * 1024 * 1024,
        ),
        cost_estimate=cost,
    )(lhs_bf, rhs_bf)


# speedup vs baseline: 1.2524x; 1.2524x over previous
import jax
import jax.numpy as jnp
from jax.experimental import pallas as pl
from jax.experimental.pallas import tpu as pltpu

_BM = 1024
_BN = 512


def _mm_body(lhs_ref, rhs_ref, out_ref):
    out_ref[...] = jnp.dot(
        lhs_ref[...].astype(jnp.bfloat16),
        rhs_ref[...].astype(jnp.bfloat16),
        preferred_element_type=jnp.float32,
    )


def kernel(lhs, rhs):
    M, K = lhs.shape
    _, N = rhs.shape

    grid = (M // _BM, N // _BN)
    cost = pl.CostEstimate(
        flops=2 * M * N * K,
        transcendentals=0,
        bytes_accessed=(M * K + K * N + M * N) * 4,
    )
    return pl.pallas_call(
        _mm_body,
        out_shape=jax.ShapeDtypeStruct((M, N), jnp.float32),
        grid=grid,
        in_specs=[
            pl.BlockSpec((_BM, K), lambda i, j: (i, 0)),
            pl.BlockSpec((K, _BN), lambda i, j: (0, j)),
        ],
        out_specs=pl.BlockSpec((_BM, _BN), lambda i, j: (i, j)),
        compiler_params=pltpu.CompilerParams(
            dimension_semantics=("parallel", "parallel"),
            vmem_limit_bytes=60 * 1024 * 1024,
        ),
        cost_estimate=cost,
    )(lhs, rhs)
